# baseline (speedup 1.0000x reference)
import math

import jax
import jax.numpy as jnp
from jax import lax
from jax.experimental import pallas as pl
from jax.experimental.pallas import tpu as pltpu
from jax.experimental.pallas import tpu_sc as plsc

RO = 4.0
N_ADJ = 4
KS = 2 * N_ADJ + 1
DILS = [1, 1, 1, 2, 2, 4, 4]
BN_INV = 1.0 / math.sqrt(1.0 + 1e-5)

B_, C_, H_, W_ = 2, 128, 256, 256
P_, N_ = 8, 128
PTS = B_ * P_ * N_
PTSB = P_ * N_
NW = 32
PPW = PTSB // NW
NG = PPW // 16


def _sc_gather_body(feat, sx, sy, out, sxv, syv, idx0, idx1,
                    rows0, rows1, sem):
    wid = lax.axis_index("s") * 2 + lax.axis_index("c")
    base = wid * PPW

    pltpu.sync_copy(sx.at[pl.ds(base, PPW)], sxv)
    pltpu.sync_copy(sy.at[pl.ds(base, PPW)], syv)

    for g in range(NG):
        xs = sxv[pl.ds(g * 16, 16)] * (1.0 / RO) - 0.5
        ys = syv[pl.ds(g * 16, 16)] * (1.0 / RO) - 0.5
        xt = xs.astype(jnp.int32)
        x0 = jnp.where(xt.astype(jnp.float32) > xs, xt - 1, xt)
        yt = ys.astype(jnp.int32)
        y0 = jnp.where(yt.astype(jnp.float32) > ys, yt - 1, yt)
        s = jnp.clip(x0, 0, W_ - 1)
        idx0[pl.ds(g * 16, 16)] = jnp.clip(y0, 0, H_ - 1) * W_ + s
        idx1[pl.ds(g * 16, 16)] = jnp.clip(y0 + 1, 0, H_ - 1) * W_ + s

    cp0 = pltpu.async_copy(feat.at[idx0], rows0, sem.at[0])
    cp1 = pltpu.async_copy(feat.at[idx1], rows1, sem.at[1])
    cp0.wait()
    cp1.wait()

    pltpu.sync_copy(rows0, out.at[pl.ds(0 * PTSB + base, PPW)])
    pltpu.sync_copy(rows1, out.at[pl.ds(1 * PTSB + base, PPW)])


def _sc_gather(feat_i32, sx, sy):
    mesh = plsc.VectorSubcoreMesh(core_axis_name="c", subcore_axis_name="s")
    return pl.kernel(
        _sc_gather_body,
        out_type=jax.ShapeDtypeStruct((2 * PTSB, C_), jnp.int32),
        mesh=mesh,
        compiler_params=pltpu.CompilerParams(needs_layout_passes=False),
        scratch_types=[
            pltpu.VMEM((PPW,), jnp.float32),
            pltpu.VMEM((PPW,), jnp.float32),
            pltpu.VMEM((PPW,), jnp.int32),
            pltpu.VMEM((PPW,), jnp.int32),
            pltpu.VMEM((PPW, C_), jnp.int32),
            pltpu.VMEM((PPW, C_), jnp.int32),
            pltpu.SemaphoreType.DMA((2,)),
        ],
    )(feat_i32, sx, sy)


YB = 8


def _tc_pack_body(feat, out):
    for i in range(YB):
        xt = jnp.swapaxes(feat[0, :, i, :], 0, 1)
        xs = jnp.concatenate([xt[1:], xt[:1]], axis=0)
        a16 = jax.lax.bitcast_convert_type(
            xt.astype(jnp.bfloat16), jnp.uint16).astype(jnp.uint32)
        b16 = jax.lax.bitcast_convert_type(
            xs.astype(jnp.bfloat16), jnp.uint16).astype(jnp.uint32)
        out[pl.ds(i * W_, W_), :] = jax.lax.bitcast_convert_type(
            a16 | (b16 << 16), jnp.int32)


def _tc_pack(feat_b):
    return pl.pallas_call(
        _tc_pack_body,
        grid=(H_ // YB,),
        in_specs=[pl.BlockSpec((1, C_, YB, W_), lambda y: (0, 0, y, 0))],
        out_specs=pl.BlockSpec((YB * W_, C_), lambda y: (y, 0)),
        out_shape=jax.ShapeDtypeStruct((H_ * W_, C_), jnp.int32),
    )(feat_b)


PB = P_
MROWS = PB * N_


def _rolled_taps(x, dil):
    c = x.shape[-1]
    x3 = x.reshape(PB, N_, c)
    taps = []
    for k in range(KS):
        s = ((k - N_ADJ) * dil) % N_
        if s == 0:
            taps.append(x)
        else:
            taps.append(
                jnp.concatenate([x3[:, s:, :], x3[:, :s, :]], axis=1)
                .reshape(MROWS, c))
    return jnp.concatenate(taps, axis=1)


def _tc_snake_body(rows2, sn, headw, resw, bias8, scale8, shift8,
                   fusw, fusb, p0w, p0b, p1w, p1b, p2w, p2b, out):
    snv = sn[:]
    sn3 = snv.reshape(PB, N_, 2)
    mins = jnp.min(sn3, axis=1, keepdims=True)
    coords = (sn3 - mins).reshape(MROWS, 2)

    x = snv[:, 0:1] * (1.0 / RO) - 0.5
    y = snv[:, 1:2] * (1.0 / RO) - 0.5
    x0 = jnp.floor(x)
    y0 = jnp.floor(y)
    fx = x - x0
    fy = y - y0
    in_x = ((x0 >= 0.0) & (x0 <= W_ - 1.0)).astype(jnp.float32)
    wa = (1.0 - fx) * in_x + fx * (x0 == -1.0).astype(jnp.float32)
    wb = fx * ((x0 >= 0.0) & (x0 <= W_ - 2.0)).astype(jnp.float32)
    wy0 = (1.0 - fy) * ((y0 >= 0.0) & (y0 <= H_ - 1.0)).astype(jnp.float32)
    y1 = y0 + 1.0
    wy1 = fy * ((y1 >= 0.0) & (y1 <= H_ - 1.0)).astype(jnp.float32)
    def unpack(v):
        u = v.astype(jnp.uint32)
        a = jax.lax.bitcast_convert_type(
            (u & 0xFFFF).astype(jnp.uint16), jnp.bfloat16)
        b = jax.lax.bitcast_convert_type(
            (u >> 16).astype(jnp.uint16), jnp.bfloat16)
        return a.astype(jnp.float32), b.astype(jnp.float32)

    a0, b0 = unpack(rows2[0])
    a1, b1 = unpack(rows2[1])
    gat_acc = ((a0 * wy0 + a1 * wy1) * wa
               + (b0 * wy0 + b1 * wy1) * wb)

    def block(x, wt, k, dil):
        s = _rolled_taps(x, dil).astype(jnp.bfloat16)
        y = jnp.dot(s, wt, preferred_element_type=jnp.float32)
        y = jnp.maximum(y + bias8[k, :].reshape(1, -1), 0.0)
        return y * scale8[k, :].reshape(1, -1) + shift8[k, :].reshape(1, -1)

    x = jnp.concatenate([gat_acc, coords], axis=1)
    x = block(x, headw[:], 0, 1)
    states = [x]
    for i, d in enumerate(DILS):
        x = block(x, resw[i], i + 1, d) + x
        states.append(x)
    state = jnp.concatenate(states, axis=1)

    state_bf = state.astype(jnp.bfloat16)
    fused = (jnp.dot(state_bf, fusw[:], preferred_element_type=jnp.float32)
             + fusb[:])
    g = jnp.max(fused.reshape(PB, N_, -1), axis=1, keepdims=True)
    gb = jnp.broadcast_to(g, (PB, N_, g.shape[-1])).reshape(MROWS, -1)
    st2 = jnp.concatenate([gb.astype(jnp.bfloat16), state_bf], axis=1)

    h = jnp.maximum(jnp.dot(st2, p0w[:], preferred_element_type=jnp.float32)
                    + p0b[:], 0.0).astype(jnp.bfloat16)
    h = jnp.maximum(jnp.dot(h, p1w[:], preferred_element_type=jnp.float32)
                    + p1b[:], 0.0).astype(jnp.bfloat16)
    off = jnp.dot(h, p2w[:], preferred_element_type=jnp.float32) + p2b[:]
    out[:] = snv + off


def _tc_snake(rows2, sn, wdict):
    full = lambda a: pl.BlockSpec(a.shape, lambda i: (0,) * a.ndim)
    row_spec = lambda a: pl.BlockSpec((MROWS,) + a.shape[1:],
                                      lambda i: (i,) + (0,) * (a.ndim - 1))
    rows2_spec = pl.BlockSpec((2, MROWS, C_), lambda i: (0, i, 0))
    ins = [rows2, sn, wdict['headw'], wdict['resw'], wdict['bias8'],
           wdict['scale8'], wdict['shift8'], wdict['fusw'], wdict['fusb'],
           wdict['p0w'], wdict['p0b'], wdict['p1w'], wdict['p1b'],
           wdict['p2w'], wdict['p2b']]
    specs = [rows2_spec, row_spec(sn)] + [full(a) for a in ins[2:]]
    return pl.pallas_call(
        _tc_snake_body,
        grid=(1,),
        in_specs=specs,
        out_specs=pl.BlockSpec((MROWS, 2), lambda i: (i, 0)),
        out_shape=jax.ShapeDtypeStruct((PTSB, 2), jnp.float32),
    )(*ins)


def _prep_weights(params):
    p = params
    w = {}
    bf = jnp.bfloat16
    w['headw'] = p['head_w'].transpose(2, 1, 0).reshape(
        KS * (C_ + 2), C_).astype(bf)
    w['resw'] = jnp.stack(
        [p['res%d_w' % i].transpose(2, 1, 0).reshape(KS * C_, C_)
         for i in range(7)]).astype(bf)
    w['bias8'] = jnp.stack([p['head_b']] + [p['res%d_b' % i] for i in range(7)])
    w['scale8'] = jnp.stack(
        [p['head_g']] + [p['res%d_g' % i] for i in range(7)]) * BN_INV
    w['shift8'] = jnp.stack(
        [p['head_bt']] + [p['res%d_bt' % i] for i in range(7)])
    w['fusw'] = p['fusion_w'][:, :, 0].T.astype(bf)
    w['fusb'] = p['fusion_b'].reshape(1, -1)
    w['p0w'] = p['p0_w'][:, :, 0].T.astype(bf)
    w['p0b'] = p['p0_b'].reshape(1, -1)
    w['p1w'] = p['p1_w'][:, :, 0].T.astype(bf)
    w['p1b'] = p['p1_b'].reshape(1, -1)
    w['p2w'] = p['p2_w'][:, :, 0].T.astype(bf)
    w['p2b'] = p['p2_b'].reshape(1, -1)
    return w


@jax.jit
def _run(cnn_feature, snakes, params):
    w = _prep_weights(params)
    sn = snakes.reshape(B_, PTSB, 2)
    outs = []
    for b in range(B_):
        feat_i32 = _tc_pack(cnn_feature[b:b + 1])
        snb = sn[b]
        rows2 = _sc_gather(feat_i32, snb[:, 0], snb[:, 1])
        outs.append(_tc_snake(rows2.reshape(2, PTSB, C_), snb, w))
    return jnp.concatenate(outs, axis=0).reshape(B_ * P_, N_, 2)


def kernel(cnn_feature, snakes, params):
    return _run(cnn_feature, snakes, params)


# speedup vs baseline: 1.4876x; 1.4876x over previous
import math

import jax
import jax.numpy as jnp
from jax import lax
from jax.experimental import pallas as pl
from jax.experimental.pallas import tpu as pltpu
from jax.experimental.pallas import tpu_sc as plsc

RO = 4.0
N_ADJ = 4
KS = 2 * N_ADJ + 1
DILS = [1, 1, 1, 2, 2, 4, 4]
BN_INV = 1.0 / math.sqrt(1.0 + 1e-5)

B_, C_, H_, W_ = 2, 128, 256, 256
P_, N_ = 8, 128
PTS = B_ * P_ * N_
NW = 32
PPW = PTS // NW
NG = PPW // 16


def _sc_gather_body(feat, sx, sy, out, sxv, syv, idx0, idx1,
                    rows0, rows1, sem):
    wid = lax.axis_index("s") * 2 + lax.axis_index("c")
    base = wid * PPW
    b_off = (base // (P_ * N_)) * (H_ * W_)

    pltpu.sync_copy(sx.at[pl.ds(base, PPW)], sxv)
    pltpu.sync_copy(sy.at[pl.ds(base, PPW)], syv)

    for g in range(NG):
        xs = sxv[pl.ds(g * 16, 16)] * (1.0 / RO) - 0.5
        ys = syv[pl.ds(g * 16, 16)] * (1.0 / RO) - 0.5
        xt = xs.astype(jnp.int32)
        x0 = jnp.where(xt.astype(jnp.float32) > xs, xt - 1, xt)
        yt = ys.astype(jnp.int32)
        y0 = jnp.where(yt.astype(jnp.float32) > ys, yt - 1, yt)
        s = jnp.clip(x0, 0, W_ - 1)
        idx0[pl.ds(g * 16, 16)] = b_off + jnp.clip(y0, 0, H_ - 1) * W_ + s
        idx1[pl.ds(g * 16, 16)] = b_off + jnp.clip(y0 + 1, 0, H_ - 1) * W_ + s

    cp0 = pltpu.async_copy(feat.at[idx0], rows0, sem.at[0])
    cp1 = pltpu.async_copy(feat.at[idx1], rows1, sem.at[1])
    cp0.wait()
    cp1.wait()

    pltpu.sync_copy(rows0, out.at[pl.ds(0 * PTS + base, PPW)])
    pltpu.sync_copy(rows1, out.at[pl.ds(1 * PTS + base, PPW)])


def _sc_gather(feat_i32, sx, sy):
    mesh = plsc.VectorSubcoreMesh(core_axis_name="c", subcore_axis_name="s")
    return pl.kernel(
        _sc_gather_body,
        out_type=jax.ShapeDtypeStruct((2 * PTS, C_), jnp.int32),
        mesh=mesh,
        compiler_params=pltpu.CompilerParams(needs_layout_passes=False),
        scratch_types=[
            pltpu.VMEM((PPW,), jnp.float32),
            pltpu.VMEM((PPW,), jnp.float32),
            pltpu.VMEM((PPW,), jnp.int32),
            pltpu.VMEM((PPW,), jnp.int32),
            pltpu.VMEM((PPW, C_), jnp.int32),
            pltpu.VMEM((PPW, C_), jnp.int32),
            pltpu.SemaphoreType.DMA((2,)),
        ],
    )(feat_i32, sx, sy)


YB = 8


def _tc_pack_body(feat, out):
    for i in range(YB):
        xt = jnp.swapaxes(feat[0, :, i, :], 0, 1)
        xs = jnp.concatenate([xt[1:], xt[:1]], axis=0)
        a16 = jax.lax.bitcast_convert_type(
            xt.astype(jnp.bfloat16), jnp.uint16).astype(jnp.uint32)
        b16 = jax.lax.bitcast_convert_type(
            xs.astype(jnp.bfloat16), jnp.uint16).astype(jnp.uint32)
        out[pl.ds(i * W_, W_), :] = jax.lax.bitcast_convert_type(
            a16 | (b16 << 16), jnp.int32)


def _tc_pack(cnn_feature):
    return pl.pallas_call(
        _tc_pack_body,
        grid=(B_, H_ // YB),
        in_specs=[pl.BlockSpec((1, C_, YB, W_), lambda b, y: (b, 0, y, 0))],
        out_specs=pl.BlockSpec((YB * W_, C_), lambda b, y: (b * (H_ // YB) + y, 0)),
        out_shape=jax.ShapeDtypeStruct((B_ * H_ * W_, C_), jnp.int32),
    )(cnn_feature)


TC_GRID = 1
PB = (B_ * P_) // TC_GRID
MROWS = PB * N_


def _rolled_taps(x, dil):
    c = x.shape[-1]
    x3 = x.reshape(PB, N_, c)
    taps = []
    for k in range(KS):
        s = ((k - N_ADJ) * dil) % N_
        if s == 0:
            taps.append(x)
        else:
            taps.append(
                jnp.concatenate([x3[:, s:, :], x3[:, :s, :]], axis=1)
                .reshape(MROWS, c))
    return jnp.concatenate(taps, axis=1)


def _tc_snake_body(rows2, sn, headw, resw, bias8, scale8, shift8,
                   fusw, fusb, p0w, p0b, p1w, p1b, p2w, p2b, out):
    snv = sn[:]
    sn3 = snv.reshape(PB, N_, 2)
    mins = jnp.min(sn3, axis=1, keepdims=True)
    coords = (sn3 - mins).reshape(MROWS, 2)

    x = snv[:, 0:1] * (1.0 / RO) - 0.5
    y = snv[:, 1:2] * (1.0 / RO) - 0.5
    x0 = jnp.floor(x)
    y0 = jnp.floor(y)
    fx = x - x0
    fy = y - y0
    in_x = ((x0 >= 0.0) & (x0 <= W_ - 1.0)).astype(jnp.float32)
    wa = (1.0 - fx) * in_x + fx * (x0 == -1.0).astype(jnp.float32)
    wb = fx * ((x0 >= 0.0) & (x0 <= W_ - 2.0)).astype(jnp.float32)
    wy0 = (1.0 - fy) * ((y0 >= 0.0) & (y0 <= H_ - 1.0)).astype(jnp.float32)
    y1 = y0 + 1.0
    wy1 = fy * ((y1 >= 0.0) & (y1 <= H_ - 1.0)).astype(jnp.float32)
    def unpack(v):
        u = v.astype(jnp.uint32)
        a = jax.lax.bitcast_convert_type(
            (u & 0xFFFF).astype(jnp.uint16), jnp.bfloat16)
        b = jax.lax.bitcast_convert_type(
            (u >> 16).astype(jnp.uint16), jnp.bfloat16)
        return a.astype(jnp.float32), b.astype(jnp.float32)

    a0, b0 = unpack(rows2[0])
    a1, b1 = unpack(rows2[1])
    gat_acc = ((a0 * wy0 + a1 * wy1) * wa
               + (b0 * wy0 + b1 * wy1) * wb)

    def block(x, wt, k, dil):
        s = _rolled_taps(x, dil).astype(jnp.bfloat16)
        y = jnp.dot(s, wt, preferred_element_type=jnp.float32)
        y = jnp.maximum(y + bias8[k, :].reshape(1, -1), 0.0)
        return y * scale8[k, :].reshape(1, -1) + shift8[k, :].reshape(1, -1)

    x = jnp.concatenate([gat_acc, coords], axis=1)
    x = block(x, headw[:], 0, 1)
    states = [x]
    for i, d in enumerate(DILS):
        x = block(x, resw[i], i + 1, d) + x
        states.append(x)
    state = jnp.concatenate(states, axis=1)

    state_bf = state.astype(jnp.bfloat16)
    fused = (jnp.dot(state_bf, fusw[:], preferred_element_type=jnp.float32)
             + fusb[:])
    g = jnp.max(fused.reshape(PB, N_, -1), axis=1, keepdims=True)
    gb = jnp.broadcast_to(g, (PB, N_, g.shape[-1])).reshape(MROWS, -1)
    st2 = jnp.concatenate([gb.astype(jnp.bfloat16), state_bf], axis=1)

    h = jnp.maximum(jnp.dot(st2, p0w[:], preferred_element_type=jnp.float32)
                    + p0b[:], 0.0).astype(jnp.bfloat16)
    h = jnp.maximum(jnp.dot(h, p1w[:], preferred_element_type=jnp.float32)
                    + p1b[:], 0.0).astype(jnp.bfloat16)
    off = jnp.dot(h, p2w[:], preferred_element_type=jnp.float32) + p2b[:]
    out[:] = snv + off


def _tc_snake(rows2, sn, wdict):
    full = lambda a: pl.BlockSpec(a.shape, lambda i: (0,) * a.ndim)
    row_spec = lambda a: pl.BlockSpec((MROWS,) + a.shape[1:],
                                      lambda i: (i,) + (0,) * (a.ndim - 1))
    rows2_spec = pl.BlockSpec((2, MROWS, C_), lambda i: (0, i, 0))
    ins = [rows2, sn, wdict['headw'], wdict['resw'], wdict['bias8'],
           wdict['scale8'], wdict['shift8'], wdict['fusw'], wdict['fusb'],
           wdict['p0w'], wdict['p0b'], wdict['p1w'], wdict['p1b'],
           wdict['p2w'], wdict['p2b']]
    specs = [rows2_spec, row_spec(sn)] + [full(a) for a in ins[2:]]
    return pl.pallas_call(
        _tc_snake_body,
        grid=(TC_GRID,),
        in_specs=specs,
        out_specs=pl.BlockSpec((MROWS, 2), lambda i: (i, 0)),
        out_shape=jax.ShapeDtypeStruct((PTS, 2), jnp.float32),
    )(*ins)


def _prep_weights(params):
    p = params
    w = {}
    bf = jnp.bfloat16
    w['headw'] = p['head_w'].transpose(2, 1, 0).reshape(
        KS * (C_ + 2), C_).astype(bf)
    w['resw'] = jnp.stack(
        [p['res%d_w' % i].transpose(2, 1, 0).reshape(KS * C_, C_)
         for i in range(7)]).astype(bf)
    w['bias8'] = jnp.stack([p['head_b']] + [p['res%d_b' % i] for i in range(7)])
    w['scale8'] = jnp.stack(
        [p['head_g']] + [p['res%d_g' % i] for i in range(7)]) * BN_INV
    w['shift8'] = jnp.stack(
        [p['head_bt']] + [p['res%d_bt' % i] for i in range(7)])
    w['fusw'] = p['fusion_w'][:, :, 0].T.astype(bf)
    w['fusb'] = p['fusion_b'].reshape(1, -1)
    w['p0w'] = p['p0_w'][:, :, 0].T.astype(bf)
    w['p0b'] = p['p0_b'].reshape(1, -1)
    w['p1w'] = p['p1_w'][:, :, 0].T.astype(bf)
    w['p1b'] = p['p1_b'].reshape(1, -1)
    w['p2w'] = p['p2_w'][:, :, 0].T.astype(bf)
    w['p2b'] = p['p2_b'].reshape(1, -1)
    return w


@jax.jit
def _run(cnn_feature, snakes, params):
    feat_i32 = _tc_pack(cnn_feature)
    sn = snakes.reshape(PTS, 2)
    rows2 = _sc_gather(feat_i32, sn[:, 0], sn[:, 1]).reshape(2, PTS, C_)
    w = _prep_weights(params)
    out = _tc_snake(rows2, sn, w)
    return out.reshape(B_ * P_, N_, 2)


def kernel(cnn_feature, snakes, params):
    return _run(cnn_feature, snakes, params)
